# baseline (speedup 1.0000x reference)
import functools
import math

import jax
import jax.numpy as jnp
from jax import lax
from jax.experimental import pallas as pl
from jax.experimental.pallas import tpu as pltpu
from jax.experimental.pallas import tpu_sc as plsc

_EMBED = 64
_ROWS = 4096
_COLS = 200
_B = _ROWS * _COLS
_NW = 32
_PER_W = _B // _NW
_CHUNK = 1600
_NCHUNK = _PER_W // _CHUNK
_SCALE = math.sqrt(_EMBED)

_mesh = plsc.VectorSubcoreMesh(core_axis_name="c", subcore_axis_name="s")


@functools.partial(
    pl.kernel,
    mesh=_mesh,
    out_type=jax.ShapeDtypeStruct((_B, _EMBED), jnp.float32),
    scratch_types=[
        pltpu.VMEM((_CHUNK,), jnp.int32),
        pltpu.VMEM((_CHUNK, _EMBED), jnp.float32),
        pltpu.SemaphoreType.DMA,
    ],
    compiler_params=pltpu.CompilerParams(use_tc_tiling_on_sc=False),
)
def _embed_lookup(x_hbm, table_hbm, out_hbm, idx_v, rows_v, sem):
    wid = lax.axis_index("s") * 2 + lax.axis_index("c")
    base = wid * _PER_W

    def chunk_body(g, carry):
        off = base + g * _CHUNK
        pltpu.sync_copy(x_hbm.at[pl.ds(off, _CHUNK)], idx_v)
        pltpu.async_copy(table_hbm.at[idx_v], rows_v, sem).wait()

        def scale_row(i, c):
            for j in range(_EMBED // 16):
                sl = pl.ds(j * 16, 16)
                rows_v[i, sl] = rows_v[i, sl] * _SCALE
            return c

        lax.fori_loop(0, _CHUNK, scale_row, 0)
        pltpu.sync_copy(rows_v, out_hbm.at[pl.ds(off, _CHUNK)])
        return carry

    lax.fori_loop(0, _NCHUNK, chunk_body, 0)


def kernel(x, table):
    out = _embed_lookup(x.reshape(_B), table)
    return out.reshape(_ROWS, _COLS, _EMBED)


# speedup vs baseline: 1.0946x; 1.0946x over previous
import functools
import math

import jax
import jax.numpy as jnp
from jax import lax
from jax.experimental import pallas as pl
from jax.experimental.pallas import tpu as pltpu
from jax.experimental.pallas import tpu_sc as plsc

_EMBED = 64
_ROWS = 4096
_COLS = 200
_B = _ROWS * _COLS
_NW = 32
_PER_W = _B // _NW
_CHUNK = 640
_NCHUNK = _PER_W // _CHUNK
_NBUF = 3
_SCALE = math.sqrt(_EMBED)

_mesh = plsc.VectorSubcoreMesh(core_axis_name="c", subcore_axis_name="s")


@functools.partial(
    pl.kernel,
    mesh=_mesh,
    out_type=jax.ShapeDtypeStruct((_B, _EMBED), jnp.float32),
    scratch_types=[
        pltpu.VMEM((_NBUF, _CHUNK), jnp.int32),
        pltpu.VMEM((_NBUF, _CHUNK, _EMBED), jnp.float32),
        pltpu.SemaphoreType.DMA((_NBUF,)),
        pltpu.SemaphoreType.DMA((_NBUF,)),
        pltpu.SemaphoreType.DMA((_NBUF,)),
    ],
    compiler_params=pltpu.CompilerParams(use_tc_tiling_on_sc=False),
)
def _embed_lookup(x_hbm, table_hbm, out_hbm, idx_v, rows_v, isem, gsem, ssem):
    wid = lax.axis_index("s") * 2 + lax.axis_index("c")
    base = wid * _PER_W

    def start_idx(k):
        slot = k % _NBUF
        return pltpu.async_copy(
            x_hbm.at[pl.ds(base + k * _CHUNK, _CHUNK)],
            idx_v.at[slot], isem.at[slot])

    def start_gather(k):
        slot = k % _NBUF
        return pltpu.async_copy(
            table_hbm.at[idx_v.at[slot]], rows_v.at[slot], gsem.at[slot])

    def start_store(k):
        slot = k % _NBUF
        return pltpu.async_copy(
            rows_v.at[slot], out_hbm.at[pl.ds(base + k * _CHUNK, _CHUNK)],
            ssem.at[slot])

    idx_cp = {}
    gather_cp = {}
    store_cp = {}

    for k in range(_NBUF):
        idx_cp[k] = start_idx(k)
    for k in range(2):
        idx_cp[k].wait()
        gather_cp[k] = start_gather(k)

    for g in range(_NCHUNK):
        slot = g % _NBUF
        gather_cp[g].wait()

        rv = rows_v.at[slot]

        def scale_row(i, c, rv=rv):
            for j in range(_EMBED // 16):
                sl = pl.ds(j * 16, 16)
                rv[i, sl] = rv[i, sl] * _SCALE
            return c

        lax.fori_loop(0, _CHUNK, scale_row, 0)
        store_cp[g] = start_store(g)

        if g + _NBUF < _NCHUNK:
            idx_cp[g + _NBUF] = start_idx(g + _NBUF)

        tg = g + 2
        if tg < _NCHUNK:
            if tg >= _NBUF:
                store_cp[tg - _NBUF].wait()
            idx_cp[tg].wait()
            gather_cp[tg] = start_gather(tg)

    for k in range(max(0, _NCHUNK - _NBUF), _NCHUNK):
        store_cp[k].wait()


def kernel(x, table):
    out = _embed_lookup(x.reshape(_B), table)
    return out.reshape(_ROWS, _COLS, _EMBED)


# speedup vs baseline: 1.0984x; 1.0034x over previous
import functools
import math

import jax
import jax.numpy as jnp
from jax import lax
from jax.experimental import pallas as pl
from jax.experimental.pallas import tpu as pltpu
from jax.experimental.pallas import tpu_sc as plsc

_EMBED = 64
_ROWS = 4096
_COLS = 200
_B = _ROWS * _COLS
_NW = 32
_XR_PER_W = _ROWS // _NW
_PER_W = _XR_PER_W * _COLS
_NBUF = 4
_NSTEP = _XR_PER_W // _NBUF
_SCALE = math.sqrt(_EMBED)

_mesh = plsc.VectorSubcoreMesh(core_axis_name="c", subcore_axis_name="s")


@functools.partial(
    pl.kernel,
    mesh=_mesh,
    out_type=jax.ShapeDtypeStruct((_ROWS, _COLS, _EMBED), jnp.float32),
    scratch_types=[
        pltpu.VMEM((_PER_W,), jnp.int32),
        pltpu.VMEM((_NBUF, _COLS, _EMBED), jnp.float32),
        pltpu.VMEM((_NBUF, _COLS, _EMBED), jnp.float32),
        pltpu.SemaphoreType.DMA((_NBUF,)),
        pltpu.SemaphoreType.DMA((_NBUF,)),
    ],
    compiler_params=pltpu.CompilerParams(use_tc_tiling_on_sc=False),
)
def _embed_lookup(x_hbm, table_hbm, out_hbm, idx_all, gbuf, sbuf, gsem, ssem):
    wid = lax.axis_index("s") * 2 + lax.axis_index("c")
    xr0 = wid * _XR_PER_W

    pltpu.sync_copy(x_hbm.at[pl.ds(wid * _PER_W, _PER_W)], idx_all)

    def gather_cp(k, b):
        return pltpu.make_async_copy(
            table_hbm.at[idx_all.at[pl.ds(k * _COLS, _COLS)]],
            gbuf.at[b], gsem.at[b])

    def store_cp(k, b):
        return pltpu.make_async_copy(
            sbuf.at[b], out_hbm.at[xr0 + k], ssem.at[b])

    for b in range(_NBUF):
        gather_cp(b, b).start()

    def step(s, carry):
        for b in range(_NBUF):
            k = s * _NBUF + b
            gather_cp(k, b).wait()

            @pl.when(s > 0)
            def _():
                store_cp(k - _NBUF, b).wait()

            def scale_row(i, c, b=b):
                for j in range(_EMBED // 16):
                    sl = pl.ds(j * 16, 16)
                    sbuf[b, i, sl] = gbuf[b, i, sl] * _SCALE
                return c

            lax.fori_loop(0, _COLS, scale_row, 0)
            store_cp(k, b).start()

            @pl.when(s < _NSTEP - 1)
            def _():
                gather_cp(k + _NBUF, b).start()

        return carry

    lax.fori_loop(0, _NSTEP, step, 0)

    for b in range(_NBUF):
        store_cp((_NSTEP - 1) * _NBUF + b, b).wait()


def kernel(x, table):
    return _embed_lookup(x.reshape(_B), table)
